# baseline (speedup 1.0000x reference)
import jax
import jax.numpy as jnp
from jax.experimental import pallas as pl
from jax.experimental.pallas import tpu as pltpu

_C = 384
_W = 64 * 64
_MUL = 1.0 / 40.0
_T_MAX = 2.0
_R = 48


def _body(a1_ref, b1_ref, a2_ref, b2_ref, out_ref):
    d = (a1_ref[...] - b1_ref[...]) * _MUL + (a2_ref[...] - b2_ref[...]) * _MUL
    out_ref[0] = jnp.minimum(d + _T_MAX, _T_MAX)
    out_ref[1] = jnp.minimum(_T_MAX - d, _T_MAX)


def kernel(tj1, tj2):
    t1 = tj1.reshape(2 * _C, _W)
    t2 = tj2.reshape(2 * _C, _W)
    nsteps = _C // _R
    top = pl.BlockSpec((_R, _W), lambda i: (i, 0))
    bot = pl.BlockSpec((_R, _W), lambda i: (i + nsteps, 0))
    out = pl.pallas_call(
        _body,
        grid=(nsteps,),
        in_specs=[top, bot, top, bot],
        out_specs=pl.BlockSpec((2, _R, _W), lambda i: (0, i, 0)),
        out_shape=jax.ShapeDtypeStruct((2, _C, _W), jnp.float32),
    )(t1, t1, t2, t2)
    return out.reshape(2 * _C, 64, 64)


# speedup vs baseline: 1.0000x; 1.0000x over previous
import jax
import jax.numpy as jnp
from jax.experimental import pallas as pl
from jax.experimental.pallas import tpu as pltpu

_C = 384
_W = 64 * 64
_MUL = 1.0 / 40.0
_T_MAX = 2.0
_R = 384


def _body(a1_ref, b1_ref, a2_ref, b2_ref, out_ref):
    d = (a1_ref[...] - b1_ref[...]) * _MUL + (a2_ref[...] - b2_ref[...]) * _MUL
    out_ref[0] = jnp.minimum(d + _T_MAX, _T_MAX)
    out_ref[1] = jnp.minimum(_T_MAX - d, _T_MAX)


def kernel(tj1, tj2):
    t1 = tj1.reshape(2 * _C, _W)
    t2 = tj2.reshape(2 * _C, _W)
    nsteps = _C // _R
    top = pl.BlockSpec((_R, _W), lambda i: (i, 0))
    bot = pl.BlockSpec((_R, _W), lambda i: (i + nsteps, 0))
    out = pl.pallas_call(
        _body,
        grid=(nsteps,),
        in_specs=[top, bot, top, bot],
        out_specs=pl.BlockSpec((2, _R, _W), lambda i: (0, i, 0)),
        out_shape=jax.ShapeDtypeStruct((2, _C, _W), jnp.float32),
    )(t1, t1, t2, t2)
    return out.reshape(2 * _C, 64, 64)


# speedup vs baseline: 1.1856x; 1.1856x over previous
import jax
import jax.numpy as jnp
from jax.experimental import pallas as pl
from jax.experimental.pallas import tpu as pltpu

_C = 384
_MUL = 1.0 / 40.0
_T_MAX = 2.0
_R = 48


def _body(a1_ref, b1_ref, a2_ref, b2_ref, out_ref):
    d = (a1_ref[...] - b1_ref[...]) * _MUL + (a2_ref[...] - b2_ref[...]) * _MUL
    out_ref[0] = jnp.minimum(d + _T_MAX, _T_MAX)
    out_ref[1] = jnp.minimum(_T_MAX - d, _T_MAX)


def kernel(tj1, tj2):
    t1 = tj1.reshape(2 * _C, 64, 64)
    t2 = tj2.reshape(2 * _C, 64, 64)
    nsteps = _C // _R
    top = pl.BlockSpec((_R, 64, 64), lambda i: (i, 0, 0))
    bot = pl.BlockSpec((_R, 64, 64), lambda i: (i + nsteps, 0, 0))
    out = pl.pallas_call(
        _body,
        grid=(nsteps,),
        in_specs=[top, bot, top, bot],
        out_specs=pl.BlockSpec((2, _R, 64, 64), lambda i: (0, i, 0, 0)),
        out_shape=jax.ShapeDtypeStruct((2, _C, 64, 64), jnp.float32),
    )(t1, t1, t2, t2)
    return out.reshape(2 * _C, 64, 64)
